# baseline (speedup 1.0000x reference)
import functools

import jax
import jax.numpy as jnp
import numpy as np
from jax import lax
from jax.experimental import pallas as pl
from jax.experimental.pallas import tpu as pltpu
from jax.experimental.pallas import tpu_sc as plsc

_H = 8
_DK = 64
_N = 8192
_E = 131072
_D = 512
_DFF = 2048

_NC = 2
_NS = 16
_NW = _NC * _NS
_NPW = _N // _NW
_GRP = 32
_NGRP = _NPW // _GRP
_CH = 64
_PAD = 256
_NV = _D // 32
_INV_SQRT_DK = 1.0 / 8.0

_BF = jnp.bfloat16
_F32 = jnp.float32


def _ln_rows(x, g, b):
    m = jnp.mean(x, axis=-1, keepdims=True)
    v = jnp.mean((x - m) ** 2, axis=-1, keepdims=True)
    return (x - m) * lax.rsqrt(v + 1e-5) * g + b


def _qkv_body(x1_ref, x2_ref, wq_ref, wk_ref, wv_ref, q_ref, k_ref, v_ref):
    x1 = x1_ref[...].astype(_BF)
    x2 = x2_ref[...].astype(_BF)
    q_ref[...] = jnp.dot(x1, wq_ref[...], preferred_element_type=_F32).astype(_BF)
    k_ref[...] = jnp.dot(x2, wk_ref[...], preferred_element_type=_F32).astype(_BF)
    v_ref[...] = jnp.dot(x2, wv_ref[...], preferred_element_type=_F32).astype(_BF)


def _qkv(x1, x2, wq, wk, wv):
    BR = 512
    return pl.pallas_call(
        _qkv_body,
        grid=(_N // BR,),
        in_specs=[
            pl.BlockSpec((BR, _D), lambda i: (i, 0)),
            pl.BlockSpec((BR, _D), lambda i: (i, 0)),
            pl.BlockSpec((_D, _D), lambda i: (0, 0)),
            pl.BlockSpec((_D, _D), lambda i: (0, 0)),
            pl.BlockSpec((_D, _D), lambda i: (0, 0)),
        ],
        out_specs=[
            pl.BlockSpec((BR, _D), lambda i: (i, 0)),
            pl.BlockSpec((BR, _D), lambda i: (i, 0)),
            pl.BlockSpec((BR, _D), lambda i: (i, 0)),
        ],
        out_shape=[jax.ShapeDtypeStruct((_N, _D), _BF)] * 3,
    )(x1, x2, wq, wk, wv)


def _kv_body(x_ref, wk_ref, wv_ref, k_ref, v_ref):
    x = x_ref[...].astype(_BF)
    k_ref[...] = jnp.dot(x, wk_ref[...], preferred_element_type=_F32).astype(_BF)
    v_ref[...] = jnp.dot(x, wv_ref[...], preferred_element_type=_F32).astype(_BF)


def _kv(x, wk, wv):
    BR = 512
    return pl.pallas_call(
        _kv_body,
        grid=(_N // BR,),
        in_specs=[
            pl.BlockSpec((BR, _D), lambda i: (i, 0)),
            pl.BlockSpec((_D, _D), lambda i: (0, 0)),
            pl.BlockSpec((_D, _D), lambda i: (0, 0)),
        ],
        out_specs=[
            pl.BlockSpec((BR, _D), lambda i: (i, 0)),
            pl.BlockSpec((BR, _D), lambda i: (i, 0)),
        ],
        out_shape=[jax.ShapeDtypeStruct((_N, _D), _BF)] * 2,
    )(x, wk, wv)


def _mm_body(x_ref, w_ref, o_ref):
    o_ref[...] = jnp.dot(
        x_ref[...].astype(_BF), w_ref[...], preferred_element_type=_F32
    ).astype(_BF)


def _mm(x, w):
    BR = 512
    return pl.pallas_call(
        _mm_body,
        grid=(_N // BR,),
        in_specs=[
            pl.BlockSpec((BR, _D), lambda i: (i, 0)),
            pl.BlockSpec((_D, _D), lambda i: (0, 0)),
        ],
        out_specs=pl.BlockSpec((BR, _D), lambda i: (i, 0)),
        out_shape=jax.ShapeDtypeStruct((_N, _D), _BF),
    )(x, w)


def _proj_res_ln_body(a_ref, wo_ref, res_ref, g_ref, b_ref, o_ref):
    y = jnp.dot(a_ref[...], wo_ref[...], preferred_element_type=_F32) + res_ref[...]
    o_ref[...] = _ln_rows(y, g_ref[...], b_ref[...])


def _proj_res_ln(a, wo, res, g, b):
    BR = 512
    return pl.pallas_call(
        _proj_res_ln_body,
        grid=(_N // BR,),
        in_specs=[
            pl.BlockSpec((BR, _D), lambda i: (i, 0)),
            pl.BlockSpec((_D, _D), lambda i: (0, 0)),
            pl.BlockSpec((BR, _D), lambda i: (i, 0)),
            pl.BlockSpec((_D,), lambda i: (0,)),
            pl.BlockSpec((_D,), lambda i: (0,)),
        ],
        out_specs=pl.BlockSpec((BR, _D), lambda i: (i, 0)),
        out_shape=jax.ShapeDtypeStruct((_N, _D), _F32),
    )(a, wo, res, g, b)


def _ffn_ln_body(x_ref, w1_ref, b1_ref, w2_ref, b2_ref, g_ref, b_ref, o_ref):
    x = x_ref[...]
    mid = jnp.dot(x.astype(_BF), w1_ref[...], preferred_element_type=_F32)
    mid = jnp.maximum(mid + b1_ref[...], 0.0).astype(_BF)
    y = x + jnp.dot(mid, w2_ref[...], preferred_element_type=_F32) + b2_ref[...]
    o_ref[...] = _ln_rows(y, g_ref[...], b_ref[...])


def _ffn_ln(x, W1, b1, W2, b2, g, b):
    BR = 512
    return pl.pallas_call(
        _ffn_ln_body,
        grid=(_N // BR,),
        in_specs=[
            pl.BlockSpec((BR, _D), lambda i: (i, 0)),
            pl.BlockSpec((_D, _DFF), lambda i: (0, 0)),
            pl.BlockSpec((_DFF,), lambda i: (0,)),
            pl.BlockSpec((_DFF, _D), lambda i: (0, 0)),
            pl.BlockSpec((_D,), lambda i: (0,)),
            pl.BlockSpec((_D,), lambda i: (0,)),
            pl.BlockSpec((_D,), lambda i: (0,)),
        ],
        out_specs=pl.BlockSpec((BR, _D), lambda i: (i, 0)),
        out_shape=jax.ShapeDtypeStruct((_N, _D), _F32),
    )(x, W1, b1, W2, b2, g, b)


def _sload(ref, idx):
    return ref[pl.ds(idx, 16)][0]


_GDN = lax.GatherDimensionNumbers(
    offset_dims=(), collapsed_slice_dims=(0,), start_index_map=(0,))


def _lane_bcast(v, lane_vec):
    return lax.gather(v, lane_vec[:, None], _GDN, slice_sizes=(1,),
                      mode=lax.GatherScatterMode.PROMISE_IN_BOUNDS)


def _edge_kernel(qa_hbm, qb_hbm, ka_hbm, kb_hbm, va_hbm, vb_hbm,
                 src_hbm, dst_hbm, oa_hbm, ob_hbm,
                 rpbuf, bsbuf, qstage, ostage, kbuf, vbuf, idxbuf, dstbuf,
                 accrow, accden, sem_k, sem_v):
    q_hbm = (qa_hbm, qb_hbm)
    k_hbm = (ka_hbm, kb_hbm)
    v_hbm = (va_hbm, vb_hbm)
    o_hbm = (oa_hbm, ob_hbm)
    wid = lax.axis_index("s") * _NC + lax.axis_index("c")
    wbase = wid * _NPW

    zero16 = jnp.zeros((16,), _F32)
    den0 = (zero16,) * _H
    iota16v = lax.broadcasted_iota(jnp.int32, (16,), 0)

    probes = jnp.minimum(wbase + _GRP * iota16v, _N)
    lo_v = jnp.zeros((16,), jnp.int32)
    hi_v = jnp.full((16,), _E, jnp.int32)

    def bs_body(t, carry):
        lo_c, hi_c = carry
        active = lo_c < hi_c
        mid = lax.shift_right_logical(lo_c + hi_c, 1)
        bsbuf[pl.ds(0, 16)] = jnp.minimum(mid, _E - 1)
        pltpu.async_copy(dst_hbm.at[bsbuf],
                         rpbuf.at[pl.ds(16, 16)], sem_k.at[0]).wait()
        dmid = rpbuf[pl.ds(16, 16)]
        go_right = jnp.logical_and(active, dmid < probes)
        go_left = jnp.logical_and(active, jnp.logical_not(dmid < probes))
        return (jnp.where(go_right, mid + 1, lo_c),
                jnp.where(go_left, mid, hi_c))

    lo_v, hi_v = lax.fori_loop(0, 18, bs_body, (lo_v, hi_v))
    rpbuf[pl.ds(0, 16)] = lo_v

    for t in range(2 * _NV):
        accrow[pl.ds(16 * t, 16)] = zero16
    for h in range(_H):
        accden[pl.ds(16 * h, 16)] = zero16

    def group_body(g, gcarry):
        gnode = wbase + g * _GRP

        pltpu.sync_copy(q_hbm[0].at[pl.ds(gnode, _GRP)], qstage.at[0])
        pltpu.sync_copy(q_hbm[1].at[pl.ds(gnode, _GRP)], qstage.at[1])

        def zrow(r, c):
            for hf in range(2):
                for i in range(8):
                    ostage[hf, r, pl.ds(16 * i, 16)] = jnp.zeros((16,), jnp.int32)
            return c
        lax.fori_loop(0, _GRP, zrow, 0)

        p_start = _sload(rpbuf, g)
        p_end = _sload(rpbuf, g + 1)
        astart = (p_start // 8) * 8
        nch = (p_end - astart + _CH - 1) // _CH

        def finalize(c):
            @pl.when(c >= 0)
            def _():
                row = c - gnode
                for h in range(_H):
                    inv_h = 1.0 / (accden[pl.ds(16 * h, 16)] + 1e-9)
                    accden[pl.ds(16 * h, 16)] = zero16
                    for i2 in (2 * h, 2 * h + 1):
                        a0 = accrow[pl.ds(32 * i2, 16)] * inv_h
                        a1 = accrow[pl.ds(32 * i2 + 16, 16)] * inv_h
                        packed = plsc.pack(a0, a1, format=plsc.PackFormat.INTERLEAVED)
                        ostage[i2 // 8, row, pl.ds(16 * (i2 % 8), 16)] = (
                            plsc.bitcast(packed, jnp.int32))
                        accrow[pl.ds(32 * i2, 16)] = zero16
                        accrow[pl.ds(32 * i2 + 16, 16)] = zero16

        lane15 = jnp.full((16,), 15, jnp.int32)

        def make_edge_helpers(p):
            def scores(j):
                d_e = _sload(dstbuf.at[p], j)
                qrow = d_e - gnode
                prods = []
                for i in range(_NV):
                    hf, off = i // 8, 16 * (i % 8)
                    kb = plsc.bitcast(kbuf[p, hf, j, pl.ds(off, 16)], _BF)
                    qb = plsc.bitcast(qstage[hf, qrow, pl.ds(off, 16)], _BF)
                    p0, p1 = plsc.unpack(kb * qb,
                                         format=plsc.PackFormat.INTERLEAVED)
                    prods.append(p0 + p1)
                exs = []
                for h in range(_H):
                    s = prods[2 * h] + prods[2 * h + 1]
                    tot_bc = _lane_bcast(plsc.cumsum(s), lane15)
                    exs.append(jnp.exp(tot_bc * _INV_SQRT_DK))
                return d_e, exs

            def apply_edge(j, d_e, exs, cur):

                def boundary(cur_):
                    finalize(cur_)
                    return d_e

                def same(cur_):
                    return cur_

                cur = lax.cond(d_e != cur, boundary, same, cur)

                for h in range(_H):
                    ex = exs[h]
                    plsc.addupdate(accden.at[pl.ds(16 * h, 16)], ex)
                    exb = plsc.pack(ex, ex, format=plsc.PackFormat.INTERLEAVED)
                    for i2 in (2 * h, 2 * h + 1):
                        vb = plsc.bitcast(
                            vbuf[p, i2 // 8, j, pl.ds(16 * (i2 % 8), 16)], _BF)
                        t0, t1 = plsc.unpack(vb * exb,
                                             format=plsc.PackFormat.INTERLEAVED)
                        plsc.addupdate(accrow.at[pl.ds(32 * i2, 16)], t0)
                        plsc.addupdate(accrow.at[pl.ds(32 * i2 + 16, 16)], t1)
                return cur

            return scores, apply_edge

        def issue(ci):
            p = jnp.bitwise_and(ci, 1)
            cstart = astart + ci * _CH
            base = jnp.minimum(cstart, _E - _CH)
            pltpu.sync_copy(src_hbm.at[pl.ds(base, _CH)], idxbuf.at[p])
            pltpu.sync_copy(dst_hbm.at[pl.ds(base, _CH)],
                            dstbuf.at[p, pl.ds(0, _CH)])
            for hf in range(2):
                pltpu.async_copy(k_hbm[hf].at[idxbuf.at[p]],
                                 kbuf.at[p, hf], sem_k.at[p])
                pltpu.async_copy(v_hbm[hf].at[idxbuf.at[p]],
                                 vbuf.at[p, hf], sem_v.at[p])

        @pl.when(nch > 0)
        def _():
            issue(jnp.int32(0))

        def chunk_body(ci, carry):
            p = jnp.bitwise_and(ci, 1)
            for hf in range(2):
                pltpu.make_async_copy(k_hbm[hf].at[idxbuf.at[p]],
                                      kbuf.at[p, hf], sem_k.at[p]).wait()
                pltpu.make_async_copy(v_hbm[hf].at[idxbuf.at[p]],
                                      vbuf.at[p, hf], sem_v.at[p]).wait()

            @pl.when(ci + 1 < nch)
            def _():
                issue(ci + 1)

            cstart = astart + ci * _CH
            base = jnp.minimum(cstart, _E - _CH)
            lo = jnp.maximum(p_start, cstart) - base
            hi = jnp.minimum(p_end, cstart + _CH) - base

            scores, apply_edge = make_edge_helpers(p)

            def one_edge(j, c):
                d_e, exs = scores(j)
                return apply_edge(j, d_e, exs, c)

            odd = jnp.bitwise_and(hi - lo, 1)
            carry = lax.cond(odd > 0, lambda c: one_edge(lo, c),
                             lambda c: c, carry)
            elo = lo + odd

            def pair_body(m, c):
                j0 = elo + 2 * m
                j1 = j0 + 1
                d0, ex0 = scores(j0)
                d1, ex1 = scores(j1)
                c = apply_edge(j0, d0, ex0, c)
                return apply_edge(j1, d1, ex1, c)

            npair = lax.shift_right_logical(hi - elo, 1)
            return lax.fori_loop(0, npair, pair_body, carry)

        cur = lax.fori_loop(0, nch, chunk_body, jnp.int32(-1))
        finalize(cur)

        pltpu.sync_copy(ostage.at[0], o_hbm[0].at[pl.ds(gnode, _GRP)])
        pltpu.sync_copy(ostage.at[1], o_hbm[1].at[pl.ds(gnode, _GRP)])
        return gcarry

    lax.fori_loop(0, _NGRP, group_body, 0)


@functools.partial(
    pl.kernel,
    out_type=[jax.ShapeDtypeStruct((_N, 128), jnp.int32)] * 2,
    mesh=plsc.VectorSubcoreMesh(core_axis_name="c", subcore_axis_name="s"),
    compiler_params=pltpu.CompilerParams(needs_layout_passes=False),
    scratch_types=[
        pltpu.VMEM((32,), jnp.int32),
        pltpu.VMEM((16,), jnp.int32),
        pltpu.VMEM((2, _GRP, 128), jnp.int32),
        pltpu.VMEM((2, _GRP, 128), jnp.int32),
        pltpu.VMEM((2, 2, _CH, 128), jnp.int32),
        pltpu.VMEM((2, 2, _CH, 128), jnp.int32),
        pltpu.VMEM((2, _CH), jnp.int32),
        pltpu.VMEM((2, _CH + 16), jnp.int32),
        pltpu.VMEM((_D,), _F32),
        pltpu.VMEM((16 * _H,), _F32),
        pltpu.SemaphoreType.DMA((2,)),
        pltpu.SemaphoreType.DMA((2,)),
    ],
)
def _edge_attn(qa, qb, ka, kb, va, vb, src_hbm, dst_hbm, oa, ob,
               rpbuf, bsbuf, qstage, ostage, kbuf, vbuf, idxbuf, dstbuf,
               accrow, accden, sem_k, sem_v):
    _edge_kernel(qa, qb, ka, kb, va, vb, src_hbm, dst_hbm, oa, ob,
                 rpbuf, bsbuf, qstage, ostage, kbuf, vbuf, idxbuf, dstbuf,
                 accrow, accden, sem_k, sem_v)


def _as_words(x):
    return lax.bitcast_convert_type(x.reshape(_N, _D // 2, 2), jnp.int32)


def _as_bf16(x):
    return lax.bitcast_convert_type(x, _BF).reshape(_N, _D)


def _edge_stage(q, k, v, src, dst):
    qw, kw, vw = _as_words(q), _as_words(k), _as_words(v)
    oa, ob = _edge_attn(qw[:, :128], qw[:, 128:], kw[:, :128], kw[:, 128:],
                        vw[:, :128], vw[:, 128:], src, dst)
    return _as_bf16(jnp.concatenate([oa, ob], axis=1))


def kernel(h, mem, edge_src, edge_dst, inter_src, inter_dst,
           Wq0, Wk0, Wv0, Wo0, Wq1, Wk1, Wv1, Wo1,
           ln0_g, ln0_b, ln1_g, ln1_b, ln2_g, ln2_b,
           W1, b1, W2, b2):
    bf = lambda w: w.astype(_BF)

    q0, k0, v0 = _qkv(h, h, bf(Wq0), bf(Wk0), bf(Wv0))
    o0 = _edge_stage(q0, k0, v0, edge_src, edge_dst)
    k1, v1 = _kv(mem, bf(Wk1), bf(Wv1))
    h1 = _proj_res_ln(o0, bf(Wo0), h, ln0_g, ln0_b)

    q1 = _mm(h1, bf(Wq1))
    o1 = _edge_stage(q1, k1, v1, inter_src, inter_dst)
    h2 = _proj_res_ln(o1, bf(Wo1), h1, ln1_g, ln1_b)

    return _ffn_ln(h2, bf(W1), b1, bf(W2), b2, ln2_g, ln2_b)


# speedup vs baseline: 1.0329x; 1.0329x over previous
import functools

import jax
import jax.numpy as jnp
import numpy as np
from jax import lax
from jax.experimental import pallas as pl
from jax.experimental.pallas import tpu as pltpu
from jax.experimental.pallas import tpu_sc as plsc

_H = 8
_DK = 64
_N = 8192
_E = 131072
_D = 512
_DFF = 2048

_NC = 2
_NS = 16
_NW = _NC * _NS
_NPW = _N // _NW
_GRP = 32
_NGRP = _NPW // _GRP
_CH = 64
_PAD = 256
_NV = _D // 32
_INV_SQRT_DK = 1.0 / 8.0

_BF = jnp.bfloat16
_F32 = jnp.float32


def _ln_rows(x, g, b):
    m = jnp.mean(x, axis=-1, keepdims=True)
    v = jnp.mean((x - m) ** 2, axis=-1, keepdims=True)
    return (x - m) * lax.rsqrt(v + 1e-5) * g + b


def _qkv_body(x1_ref, x2_ref, wq_ref, wk_ref, wv_ref, q_ref, k_ref, v_ref):
    x1 = x1_ref[...].astype(_BF)
    x2 = x2_ref[...].astype(_BF)
    q_ref[...] = jnp.dot(x1, wq_ref[...], preferred_element_type=_F32).astype(_BF)
    k_ref[...] = jnp.dot(x2, wk_ref[...], preferred_element_type=_F32).astype(_BF)
    v_ref[...] = jnp.dot(x2, wv_ref[...], preferred_element_type=_F32).astype(_BF)


def _qkv(x1, x2, wq, wk, wv):
    BR = 512
    return pl.pallas_call(
        _qkv_body,
        grid=(_N // BR,),
        in_specs=[
            pl.BlockSpec((BR, _D), lambda i: (i, 0)),
            pl.BlockSpec((BR, _D), lambda i: (i, 0)),
            pl.BlockSpec((_D, _D), lambda i: (0, 0)),
            pl.BlockSpec((_D, _D), lambda i: (0, 0)),
            pl.BlockSpec((_D, _D), lambda i: (0, 0)),
        ],
        out_specs=[
            pl.BlockSpec((BR, _D), lambda i: (i, 0)),
            pl.BlockSpec((BR, _D), lambda i: (i, 0)),
            pl.BlockSpec((BR, _D), lambda i: (i, 0)),
        ],
        out_shape=[jax.ShapeDtypeStruct((_N, _D), _BF)] * 3,
    )(x1, x2, wq, wk, wv)


def _kv_body(x_ref, wk_ref, wv_ref, k_ref, v_ref):
    x = x_ref[...].astype(_BF)
    k_ref[...] = jnp.dot(x, wk_ref[...], preferred_element_type=_F32).astype(_BF)
    v_ref[...] = jnp.dot(x, wv_ref[...], preferred_element_type=_F32).astype(_BF)


def _kv(x, wk, wv):
    BR = 512
    return pl.pallas_call(
        _kv_body,
        grid=(_N // BR,),
        in_specs=[
            pl.BlockSpec((BR, _D), lambda i: (i, 0)),
            pl.BlockSpec((_D, _D), lambda i: (0, 0)),
            pl.BlockSpec((_D, _D), lambda i: (0, 0)),
        ],
        out_specs=[
            pl.BlockSpec((BR, _D), lambda i: (i, 0)),
            pl.BlockSpec((BR, _D), lambda i: (i, 0)),
        ],
        out_shape=[jax.ShapeDtypeStruct((_N, _D), _BF)] * 2,
    )(x, wk, wv)


def _mm_body(x_ref, w_ref, o_ref):
    o_ref[...] = jnp.dot(
        x_ref[...].astype(_BF), w_ref[...], preferred_element_type=_F32
    ).astype(_BF)


def _mm(x, w):
    BR = 512
    return pl.pallas_call(
        _mm_body,
        grid=(_N // BR,),
        in_specs=[
            pl.BlockSpec((BR, _D), lambda i: (i, 0)),
            pl.BlockSpec((_D, _D), lambda i: (0, 0)),
        ],
        out_specs=pl.BlockSpec((BR, _D), lambda i: (i, 0)),
        out_shape=jax.ShapeDtypeStruct((_N, _D), _BF),
    )(x, w)


def _proj_res_ln_body(a_ref, wo_ref, res_ref, g_ref, b_ref, o_ref):
    y = jnp.dot(a_ref[...], wo_ref[...], preferred_element_type=_F32) + res_ref[...]
    o_ref[...] = _ln_rows(y, g_ref[...], b_ref[...])


def _proj_res_ln(a, wo, res, g, b):
    BR = 512
    return pl.pallas_call(
        _proj_res_ln_body,
        grid=(_N // BR,),
        in_specs=[
            pl.BlockSpec((BR, _D), lambda i: (i, 0)),
            pl.BlockSpec((_D, _D), lambda i: (0, 0)),
            pl.BlockSpec((BR, _D), lambda i: (i, 0)),
            pl.BlockSpec((_D,), lambda i: (0,)),
            pl.BlockSpec((_D,), lambda i: (0,)),
        ],
        out_specs=pl.BlockSpec((BR, _D), lambda i: (i, 0)),
        out_shape=jax.ShapeDtypeStruct((_N, _D), _F32),
    )(a, wo, res, g, b)


def _ffn_ln_body(x_ref, w1_ref, b1_ref, w2_ref, b2_ref, g_ref, b_ref, o_ref):
    x = x_ref[...]
    mid = jnp.dot(x.astype(_BF), w1_ref[...], preferred_element_type=_F32)
    mid = jnp.maximum(mid + b1_ref[...], 0.0).astype(_BF)
    y = x + jnp.dot(mid, w2_ref[...], preferred_element_type=_F32) + b2_ref[...]
    o_ref[...] = _ln_rows(y, g_ref[...], b_ref[...])


def _ffn_ln(x, W1, b1, W2, b2, g, b):
    BR = 512
    return pl.pallas_call(
        _ffn_ln_body,
        grid=(_N // BR,),
        in_specs=[
            pl.BlockSpec((BR, _D), lambda i: (i, 0)),
            pl.BlockSpec((_D, _DFF), lambda i: (0, 0)),
            pl.BlockSpec((_DFF,), lambda i: (0,)),
            pl.BlockSpec((_DFF, _D), lambda i: (0, 0)),
            pl.BlockSpec((_D,), lambda i: (0,)),
            pl.BlockSpec((_D,), lambda i: (0,)),
            pl.BlockSpec((_D,), lambda i: (0,)),
        ],
        out_specs=pl.BlockSpec((BR, _D), lambda i: (i, 0)),
        out_shape=jax.ShapeDtypeStruct((_N, _D), _F32),
    )(x, W1, b1, W2, b2, g, b)


def _sload(ref, idx):
    return ref[pl.ds(idx, 16)][0]


_GDN = lax.GatherDimensionNumbers(
    offset_dims=(), collapsed_slice_dims=(0,), start_index_map=(0,))


def _lane_bcast(v, lane_vec):
    return lax.gather(v, lane_vec[:, None], _GDN, slice_sizes=(1,),
                      mode=lax.GatherScatterMode.PROMISE_IN_BOUNDS)


def _edge_kernel(qa_hbm, qb_hbm, ka_hbm, kb_hbm, va_hbm, vb_hbm,
                 src_hbm, dst_hbm, oa_hbm, ob_hbm,
                 rpbuf, bsbuf, qstage, ostage, kbuf, vbuf, idxbuf, dstbuf,
                 accrow, sem_k, sem_v):
    q_hbm = (qa_hbm, qb_hbm)
    k_hbm = (ka_hbm, kb_hbm)
    v_hbm = (va_hbm, vb_hbm)
    o_hbm = (oa_hbm, ob_hbm)
    wid = lax.axis_index("s") * _NC + lax.axis_index("c")
    wbase = wid * _NPW

    zero16 = jnp.zeros((16,), _F32)
    den0 = (zero16,) * _H
    iota16v = lax.broadcasted_iota(jnp.int32, (16,), 0)

    probes = jnp.minimum(wbase + _GRP * iota16v, _N)
    lo_v = jnp.zeros((16,), jnp.int32)
    hi_v = jnp.full((16,), _E, jnp.int32)

    def bs_body(t, carry):
        lo_c, hi_c = carry
        active = lo_c < hi_c
        mid = lax.shift_right_logical(lo_c + hi_c, 1)
        bsbuf[pl.ds(0, 16)] = jnp.minimum(mid, _E - 1)
        pltpu.async_copy(dst_hbm.at[bsbuf],
                         rpbuf.at[pl.ds(16, 16)], sem_k.at[0]).wait()
        dmid = rpbuf[pl.ds(16, 16)]
        go_right = jnp.logical_and(active, dmid < probes)
        go_left = jnp.logical_and(active, jnp.logical_not(dmid < probes))
        return (jnp.where(go_right, mid + 1, lo_c),
                jnp.where(go_left, mid, hi_c))

    lo_v, hi_v = lax.fori_loop(0, 18, bs_body, (lo_v, hi_v))
    rpbuf[pl.ds(0, 16)] = lo_v

    for t in range(2 * _NV):
        accrow[pl.ds(16 * t, 16)] = zero16

    def group_body(g, gcarry):
        gnode = wbase + g * _GRP

        pltpu.sync_copy(q_hbm[0].at[pl.ds(gnode, _GRP)], qstage.at[0])
        pltpu.sync_copy(q_hbm[1].at[pl.ds(gnode, _GRP)], qstage.at[1])

        def zrow(r, c):
            for hf in range(2):
                for i in range(8):
                    ostage[hf, r, pl.ds(16 * i, 16)] = jnp.zeros((16,), jnp.int32)
            return c
        lax.fori_loop(0, _GRP, zrow, 0)

        p_start = _sload(rpbuf, g)
        p_end = _sload(rpbuf, g + 1)
        astart = (p_start // 8) * 8
        nch = (p_end - astart + _CH - 1) // _CH

        def finalize(c, dn):
            @pl.when(c >= 0)
            def _():
                row = c - gnode
                for h in range(_H):
                    inv_h = 1.0 / (dn[h] + 1e-9)
                    for i2 in (2 * h, 2 * h + 1):
                        a0 = accrow[pl.ds(32 * i2, 16)] * inv_h
                        a1 = accrow[pl.ds(32 * i2 + 16, 16)] * inv_h
                        packed = plsc.pack(a0, a1, format=plsc.PackFormat.INTERLEAVED)
                        ostage[i2 // 8, row, pl.ds(16 * (i2 % 8), 16)] = (
                            plsc.bitcast(packed, jnp.int32))
                        accrow[pl.ds(32 * i2, 16)] = zero16
                        accrow[pl.ds(32 * i2 + 16, 16)] = zero16

        lane15 = jnp.full((16,), 15, jnp.int32)

        def make_edge_helpers(p):
            def scores(j):
                d_e = _sload(dstbuf.at[p], j)
                qrow = d_e - gnode
                prods = []
                for i in range(_NV):
                    hf, off = i // 8, 16 * (i % 8)
                    kb = plsc.bitcast(kbuf[p, hf, j, pl.ds(off, 16)], _BF)
                    qb = plsc.bitcast(qstage[hf, qrow, pl.ds(off, 16)], _BF)
                    p0, p1 = plsc.unpack(kb * qb,
                                         format=plsc.PackFormat.INTERLEAVED)
                    prods.append(p0 + p1)
                exs = []
                for h in range(_H):
                    s = prods[2 * h] + prods[2 * h + 1]
                    tot_bc = _lane_bcast(plsc.cumsum(s), lane15)
                    exs.append(jnp.exp(tot_bc * _INV_SQRT_DK))
                return d_e, exs

            def apply_edge(j, d_e, exs, carry):
                cur, den = carry

                def boundary(cur_, den_):
                    finalize(cur_, den_)
                    return d_e, den0

                def same(cur_, den_):
                    return cur_, den_

                cur, den = lax.cond(d_e != cur, boundary, same, cur, den)

                new_den = []
                for h in range(_H):
                    ex = exs[h]
                    new_den.append(den[h] + ex)
                    exb = plsc.pack(ex, ex, format=plsc.PackFormat.INTERLEAVED)
                    for i2 in (2 * h, 2 * h + 1):
                        vb = plsc.bitcast(
                            vbuf[p, i2 // 8, j, pl.ds(16 * (i2 % 8), 16)], _BF)
                        t0, t1 = plsc.unpack(vb * exb,
                                             format=plsc.PackFormat.INTERLEAVED)
                        plsc.addupdate(accrow.at[pl.ds(32 * i2, 16)], t0)
                        plsc.addupdate(accrow.at[pl.ds(32 * i2 + 16, 16)], t1)
                return cur, tuple(new_den)

            return scores, apply_edge

        def issue(ci):
            p = jnp.bitwise_and(ci, 1)
            cstart = astart + ci * _CH
            base = jnp.minimum(cstart, _E - _CH)
            pltpu.sync_copy(src_hbm.at[pl.ds(base, _CH)], idxbuf.at[p])
            pltpu.sync_copy(dst_hbm.at[pl.ds(base, _CH)],
                            dstbuf.at[p, pl.ds(0, _CH)])
            for hf in range(2):
                pltpu.async_copy(k_hbm[hf].at[idxbuf.at[p]],
                                 kbuf.at[p, hf], sem_k.at[p])
                pltpu.async_copy(v_hbm[hf].at[idxbuf.at[p]],
                                 vbuf.at[p, hf], sem_v.at[p])

        @pl.when(nch > 0)
        def _():
            issue(jnp.int32(0))

        def chunk_body(ci, carry):
            p = jnp.bitwise_and(ci, 1)
            for hf in range(2):
                pltpu.make_async_copy(k_hbm[hf].at[idxbuf.at[p]],
                                      kbuf.at[p, hf], sem_k.at[p]).wait()
                pltpu.make_async_copy(v_hbm[hf].at[idxbuf.at[p]],
                                      vbuf.at[p, hf], sem_v.at[p]).wait()

            @pl.when(ci + 1 < nch)
            def _():
                issue(ci + 1)

            cstart = astart + ci * _CH
            base = jnp.minimum(cstart, _E - _CH)
            lo = jnp.maximum(p_start, cstart) - base
            hi = jnp.minimum(p_end, cstart + _CH) - base

            scores, apply_edge = make_edge_helpers(p)

            def one_edge(j, c):
                d_e, exs = scores(j)
                return apply_edge(j, d_e, exs, c)

            odd = jnp.bitwise_and(hi - lo, 1)
            carry = lax.cond(odd > 0, lambda c: one_edge(lo, c),
                             lambda c: c, carry)
            elo = lo + odd

            def pair_body(m, c):
                j0 = elo + 2 * m
                j1 = j0 + 1
                d0, ex0 = scores(j0)
                d1, ex1 = scores(j1)
                c = apply_edge(j0, d0, ex0, c)
                return apply_edge(j1, d1, ex1, c)

            npair = lax.shift_right_logical(hi - elo, 1)
            return lax.fori_loop(0, npair, pair_body, carry)

        cur, den = lax.fori_loop(0, nch, chunk_body, (jnp.int32(-1), den0))
        finalize(cur, den)

        pltpu.sync_copy(ostage.at[0], o_hbm[0].at[pl.ds(gnode, _GRP)])
        pltpu.sync_copy(ostage.at[1], o_hbm[1].at[pl.ds(gnode, _GRP)])
        return gcarry

    lax.fori_loop(0, _NGRP, group_body, 0)


@functools.partial(
    pl.kernel,
    out_type=[jax.ShapeDtypeStruct((_N, 128), jnp.int32)] * 2,
    mesh=plsc.VectorSubcoreMesh(core_axis_name="c", subcore_axis_name="s"),
    compiler_params=pltpu.CompilerParams(needs_layout_passes=False),
    scratch_types=[
        pltpu.VMEM((32,), jnp.int32),
        pltpu.VMEM((16,), jnp.int32),
        pltpu.VMEM((2, _GRP, 128), jnp.int32),
        pltpu.VMEM((2, _GRP, 128), jnp.int32),
        pltpu.VMEM((2, 2, _CH, 128), jnp.int32),
        pltpu.VMEM((2, 2, _CH, 128), jnp.int32),
        pltpu.VMEM((2, _CH), jnp.int32),
        pltpu.VMEM((2, _CH + 16), jnp.int32),
        pltpu.VMEM((_D,), _F32),
        pltpu.SemaphoreType.DMA((2,)),
        pltpu.SemaphoreType.DMA((2,)),
    ],
)
def _edge_attn(qa, qb, ka, kb, va, vb, src_hbm, dst_hbm, oa, ob,
               rpbuf, bsbuf, qstage, ostage, kbuf, vbuf, idxbuf, dstbuf,
               accrow, sem_k, sem_v):
    _edge_kernel(qa, qb, ka, kb, va, vb, src_hbm, dst_hbm, oa, ob,
                 rpbuf, bsbuf, qstage, ostage, kbuf, vbuf, idxbuf, dstbuf,
                 accrow, sem_k, sem_v)


def _as_words(x):
    return lax.bitcast_convert_type(x.reshape(_N, _D // 2, 2), jnp.int32)


def _as_bf16(x):
    return lax.bitcast_convert_type(x, _BF).reshape(_N, _D)


def _edge_stage(q, k, v, src, dst):
    qw, kw, vw = _as_words(q), _as_words(k), _as_words(v)
    oa, ob = _edge_attn(qw[:, :128], qw[:, 128:], kw[:, :128], kw[:, 128:],
                        vw[:, :128], vw[:, 128:], src, dst)
    return _as_bf16(jnp.concatenate([oa, ob], axis=1))


def kernel(h, mem, edge_src, edge_dst, inter_src, inter_dst,
           Wq0, Wk0, Wv0, Wo0, Wq1, Wk1, Wv1, Wo1,
           ln0_g, ln0_b, ln1_g, ln1_b, ln2_g, ln2_b,
           W1, b1, W2, b2):
    bf = lambda w: w.astype(_BF)

    q0, k0, v0 = _qkv(h, h, bf(Wq0), bf(Wk0), bf(Wv0))
    o0 = _edge_stage(q0, k0, v0, edge_src, edge_dst)
    k1, v1 = _kv(mem, bf(Wk1), bf(Wv1))
    h1 = _proj_res_ln(o0, bf(Wo0), h, ln0_g, ln0_b)

    q1 = _mm(h1, bf(Wq1))
    o1 = _edge_stage(q1, k1, v1, inter_src, inter_dst)
    h2 = _proj_res_ln(o1, bf(Wo1), h1, ln1_g, ln1_b)

    return _ffn_ln(h2, bf(W1), b1, bf(W2), b2, ln2_g, ln2_b)


# speedup vs baseline: 1.0412x; 1.0080x over previous
import functools

import jax
import jax.numpy as jnp
from jax import lax
from jax.experimental import pallas as pl
from jax.experimental.pallas import tpu as pltpu
from jax.experimental.pallas import tpu_sc as plsc

_H = 8
_DK = 64
_N = 8192
_E = 131072
_D = 512
_DFF = 2048

_NC = 2
_NS = 16
_NW = _NC * _NS
_NPW = _N // _NW
_GRP = 32
_NGRP = _NPW // _GRP
_CH = 96
_NV = _D // 32
_INV_SQRT_DK = 1.0 / 8.0

_BF = jnp.bfloat16
_F32 = jnp.float32


def _ln_rows(x, g, b):
    m = jnp.mean(x, axis=-1, keepdims=True)
    v = jnp.mean((x - m) ** 2, axis=-1, keepdims=True)
    return (x - m) * lax.rsqrt(v + 1e-5) * g + b


def _qkv_body(x1_ref, x2_ref, wq_ref, wk_ref, wv_ref, q_ref, k_ref, v_ref):
    x1 = x1_ref[...].astype(_BF)
    x2 = x2_ref[...].astype(_BF)
    q_ref[...] = jnp.dot(x1, wq_ref[...], preferred_element_type=_F32).astype(_BF)
    k_ref[...] = jnp.dot(x2, wk_ref[...], preferred_element_type=_F32).astype(_BF)
    v_ref[...] = jnp.dot(x2, wv_ref[...], preferred_element_type=_F32).astype(_BF)


def _qkv(x1, x2, wq, wk, wv):
    BR = 512
    return pl.pallas_call(
        _qkv_body,
        grid=(_N // BR,),
        in_specs=[
            pl.BlockSpec((BR, _D), lambda i: (i, 0)),
            pl.BlockSpec((BR, _D), lambda i: (i, 0)),
            pl.BlockSpec((_D, _D), lambda i: (0, 0)),
            pl.BlockSpec((_D, _D), lambda i: (0, 0)),
            pl.BlockSpec((_D, _D), lambda i: (0, 0)),
        ],
        out_specs=[
            pl.BlockSpec((BR, _D), lambda i: (i, 0)),
            pl.BlockSpec((BR, _D), lambda i: (i, 0)),
            pl.BlockSpec((BR, _D), lambda i: (i, 0)),
        ],
        out_shape=[jax.ShapeDtypeStruct((_N, _D), _BF)] * 3,
    )(x1, x2, wq, wk, wv)


def _kv_body(x_ref, wk_ref, wv_ref, k_ref, v_ref):
    x = x_ref[...].astype(_BF)
    k_ref[...] = jnp.dot(x, wk_ref[...], preferred_element_type=_F32).astype(_BF)
    v_ref[...] = jnp.dot(x, wv_ref[...], preferred_element_type=_F32).astype(_BF)


def _kv(x, wk, wv):
    BR = 512
    return pl.pallas_call(
        _kv_body,
        grid=(_N // BR,),
        in_specs=[
            pl.BlockSpec((BR, _D), lambda i: (i, 0)),
            pl.BlockSpec((_D, _D), lambda i: (0, 0)),
            pl.BlockSpec((_D, _D), lambda i: (0, 0)),
        ],
        out_specs=[
            pl.BlockSpec((BR, _D), lambda i: (i, 0)),
            pl.BlockSpec((BR, _D), lambda i: (i, 0)),
        ],
        out_shape=[jax.ShapeDtypeStruct((_N, _D), _BF)] * 2,
    )(x, wk, wv)


def _mm_body(x_ref, w_ref, o_ref):
    o_ref[...] = jnp.dot(
        x_ref[...].astype(_BF), w_ref[...], preferred_element_type=_F32
    ).astype(_BF)


def _mm(x, w):
    BR = 512
    return pl.pallas_call(
        _mm_body,
        grid=(_N // BR,),
        in_specs=[
            pl.BlockSpec((BR, _D), lambda i: (i, 0)),
            pl.BlockSpec((_D, _D), lambda i: (0, 0)),
        ],
        out_specs=pl.BlockSpec((BR, _D), lambda i: (i, 0)),
        out_shape=jax.ShapeDtypeStruct((_N, _D), _BF),
    )(x, w)


def _proj_res_ln_body(a_ref, wo_ref, res_ref, g_ref, b_ref, o_ref):
    y = jnp.dot(a_ref[...], wo_ref[...], preferred_element_type=_F32) + res_ref[...]
    o_ref[...] = _ln_rows(y, g_ref[...], b_ref[...])


def _proj_res_ln(a, wo, res, g, b):
    BR = 512
    return pl.pallas_call(
        _proj_res_ln_body,
        grid=(_N // BR,),
        in_specs=[
            pl.BlockSpec((BR, _D), lambda i: (i, 0)),
            pl.BlockSpec((_D, _D), lambda i: (0, 0)),
            pl.BlockSpec((BR, _D), lambda i: (i, 0)),
            pl.BlockSpec((_D,), lambda i: (0,)),
            pl.BlockSpec((_D,), lambda i: (0,)),
        ],
        out_specs=pl.BlockSpec((BR, _D), lambda i: (i, 0)),
        out_shape=jax.ShapeDtypeStruct((_N, _D), _F32),
    )(a, wo, res, g, b)


def _ffn_ln_body(x_ref, w1_ref, b1_ref, w2_ref, b2_ref, g_ref, b_ref, o_ref):
    x = x_ref[...]
    mid = jnp.dot(x.astype(_BF), w1_ref[...], preferred_element_type=_F32)
    mid = jnp.maximum(mid + b1_ref[...], 0.0).astype(_BF)
    y = x + jnp.dot(mid, w2_ref[...], preferred_element_type=_F32) + b2_ref[...]
    o_ref[...] = _ln_rows(y, g_ref[...], b_ref[...])


def _ffn_ln(x, W1, b1, W2, b2, g, b):
    BR = 512
    return pl.pallas_call(
        _ffn_ln_body,
        grid=(_N // BR,),
        in_specs=[
            pl.BlockSpec((BR, _D), lambda i: (i, 0)),
            pl.BlockSpec((_D, _DFF), lambda i: (0, 0)),
            pl.BlockSpec((_DFF,), lambda i: (0,)),
            pl.BlockSpec((_DFF, _D), lambda i: (0, 0)),
            pl.BlockSpec((_D,), lambda i: (0,)),
            pl.BlockSpec((_D,), lambda i: (0,)),
            pl.BlockSpec((_D,), lambda i: (0,)),
        ],
        out_specs=pl.BlockSpec((BR, _D), lambda i: (i, 0)),
        out_shape=jax.ShapeDtypeStruct((_N, _D), _F32),
    )(x, W1, b1, W2, b2, g, b)


def _sload(ref, idx):
    return ref[pl.ds(idx, 16)][0]


_GDN = lax.GatherDimensionNumbers(
    offset_dims=(), collapsed_slice_dims=(0,), start_index_map=(0,))


def _lane_bcast(v, lane_vec):
    return lax.gather(v, lane_vec[:, None], _GDN, slice_sizes=(1,),
                      mode=lax.GatherScatterMode.PROMISE_IN_BOUNDS)


def _edge_kernel(qa_hbm, qb_hbm, ka_hbm, kb_hbm, va_hbm, vb_hbm,
                 src_hbm, dst_hbm, oa_hbm, ob_hbm,
                 rpbuf, bsbuf, qstage, ostage, kbuf, vbuf, idxbuf, dstbuf,
                 accrow, sem_k, sem_v):
    q_hbm = (qa_hbm, qb_hbm)
    k_hbm = (ka_hbm, kb_hbm)
    v_hbm = (va_hbm, vb_hbm)
    o_hbm = (oa_hbm, ob_hbm)
    wid = lax.axis_index("s") * _NC + lax.axis_index("c")
    wbase = wid * _NPW

    zero16 = jnp.zeros((16,), _F32)
    den0 = (zero16,) * _H
    iota16v = lax.broadcasted_iota(jnp.int32, (16,), 0)

    probes = jnp.minimum(wbase + _GRP * iota16v, _N)
    lo_v = jnp.zeros((16,), jnp.int32)
    hi_v = jnp.full((16,), _E, jnp.int32)

    def bs_body(t, carry):
        lo_c, hi_c = carry
        active = lo_c < hi_c
        mid = lax.shift_right_logical(lo_c + hi_c, 1)
        bsbuf[pl.ds(0, 16)] = jnp.minimum(mid, _E - 1)
        pltpu.async_copy(dst_hbm.at[bsbuf],
                         rpbuf.at[pl.ds(16, 16)], sem_k.at[0]).wait()
        dmid = rpbuf[pl.ds(16, 16)]
        go_right = jnp.logical_and(active, dmid < probes)
        go_left = jnp.logical_and(active, jnp.logical_not(dmid < probes))
        return (jnp.where(go_right, mid + 1, lo_c),
                jnp.where(go_left, mid, hi_c))

    lo_v, hi_v = lax.fori_loop(0, 18, bs_body, (lo_v, hi_v))
    rpbuf[pl.ds(0, 16)] = lo_v

    for t in range(2 * _NV):
        accrow[pl.ds(16 * t, 16)] = zero16

    def group_body(g, gcarry):
        gnode = wbase + g * _GRP

        pltpu.sync_copy(q_hbm[0].at[pl.ds(gnode, _GRP)], qstage.at[0])
        pltpu.sync_copy(q_hbm[1].at[pl.ds(gnode, _GRP)], qstage.at[1])

        def zrow(r, c):
            for hf in range(2):
                for i in range(8):
                    ostage[hf, r, pl.ds(16 * i, 16)] = jnp.zeros((16,), jnp.int32)
            return c
        lax.fori_loop(0, _GRP, zrow, 0)

        p_start = _sload(rpbuf, g)
        p_end = _sload(rpbuf, g + 1)
        astart = (p_start // 8) * 8
        nch = (p_end - astart + _CH - 1) // _CH

        def finalize(c, dn):
            @pl.when(c >= 0)
            def _():
                row = c - gnode
                for h in range(_H):
                    inv_h = 1.0 / (dn[h] + 1e-9)
                    for i2 in (2 * h, 2 * h + 1):
                        a0 = accrow[pl.ds(32 * i2, 16)] * inv_h
                        a1 = accrow[pl.ds(32 * i2 + 16, 16)] * inv_h
                        packed = plsc.pack(a0, a1, format=plsc.PackFormat.INTERLEAVED)
                        ostage[i2 // 8, row, pl.ds(16 * (i2 % 8), 16)] = (
                            plsc.bitcast(packed, jnp.int32))
                        accrow[pl.ds(32 * i2, 16)] = zero16
                        accrow[pl.ds(32 * i2 + 16, 16)] = zero16

        lane15 = jnp.full((16,), 15, jnp.int32)

        def make_edge_helpers(p):
            def scores(j):
                d_e = _sload(dstbuf.at[p], j)
                qrow = d_e - gnode
                prods = []
                for i in range(_NV):
                    hf, off = i // 8, 16 * (i % 8)
                    kb = plsc.bitcast(kbuf[p, hf, j, pl.ds(off, 16)], _BF)
                    qb = plsc.bitcast(qstage[hf, qrow, pl.ds(off, 16)], _BF)
                    p0, p1 = plsc.unpack(kb * qb,
                                         format=plsc.PackFormat.INTERLEAVED)
                    prods.append(p0 + p1)
                exs = []
                for h in range(_H):
                    s = prods[2 * h] + prods[2 * h + 1]
                    tot_bc = _lane_bcast(plsc.cumsum(s), lane15)
                    exs.append(jnp.exp(tot_bc * _INV_SQRT_DK))
                return d_e, exs

            def apply_edge(j, d_e, exs, carry):
                cur, den = carry

                def boundary(cur_, den_):
                    finalize(cur_, den_)
                    return d_e, den0

                def same(cur_, den_):
                    return cur_, den_

                cur, den = lax.cond(d_e != cur, boundary, same, cur, den)

                new_den = []
                for h in range(_H):
                    ex = exs[h]
                    new_den.append(den[h] + ex)
                    exb = plsc.pack(ex, ex, format=plsc.PackFormat.INTERLEAVED)
                    for i2 in (2 * h, 2 * h + 1):
                        vb = plsc.bitcast(
                            vbuf[p, i2 // 8, j, pl.ds(16 * (i2 % 8), 16)], _BF)
                        t0, t1 = plsc.unpack(vb * exb,
                                             format=plsc.PackFormat.INTERLEAVED)
                        plsc.addupdate(accrow.at[pl.ds(32 * i2, 16)], t0)
                        plsc.addupdate(accrow.at[pl.ds(32 * i2 + 16, 16)], t1)
                return cur, tuple(new_den)

            return scores, apply_edge

        def issue(ci):
            p = jnp.bitwise_and(ci, 1)
            cstart = astart + ci * _CH
            base = jnp.minimum(cstart, _E - _CH)
            pltpu.sync_copy(src_hbm.at[pl.ds(base, _CH)], idxbuf.at[p])
            pltpu.sync_copy(dst_hbm.at[pl.ds(base, _CH)],
                            dstbuf.at[p, pl.ds(0, _CH)])
            for hf in range(2):
                pltpu.async_copy(k_hbm[hf].at[idxbuf.at[p]],
                                 kbuf.at[p, hf], sem_k.at[p])
                pltpu.async_copy(v_hbm[hf].at[idxbuf.at[p]],
                                 vbuf.at[p, hf], sem_v.at[p])

        @pl.when(nch > 0)
        def _():
            issue(jnp.int32(0))

        def chunk_body(ci, carry):
            p = jnp.bitwise_and(ci, 1)
            for hf in range(2):
                pltpu.make_async_copy(k_hbm[hf].at[idxbuf.at[p]],
                                      kbuf.at[p, hf], sem_k.at[p]).wait()
                pltpu.make_async_copy(v_hbm[hf].at[idxbuf.at[p]],
                                      vbuf.at[p, hf], sem_v.at[p]).wait()

            @pl.when(ci + 1 < nch)
            def _():
                issue(ci + 1)

            cstart = astart + ci * _CH
            base = jnp.minimum(cstart, _E - _CH)
            lo = jnp.maximum(p_start, cstart) - base
            hi = jnp.minimum(p_end, cstart + _CH) - base

            scores, apply_edge = make_edge_helpers(p)

            def one_edge(j, c):
                d_e, exs = scores(j)
                return apply_edge(j, d_e, exs, c)

            odd = jnp.bitwise_and(hi - lo, 1)
            carry = lax.cond(odd > 0, lambda c: one_edge(lo, c),
                             lambda c: c, carry)
            elo = lo + odd

            def pair_body(m, c):
                j0 = elo + 2 * m
                j1 = j0 + 1
                d0, ex0 = scores(j0)
                d1, ex1 = scores(j1)
                c = apply_edge(j0, d0, ex0, c)
                return apply_edge(j1, d1, ex1, c)

            npair = lax.shift_right_logical(hi - elo, 1)
            return lax.fori_loop(0, npair, pair_body, carry)

        cur, den = lax.fori_loop(0, nch, chunk_body, (jnp.int32(-1), den0))
        finalize(cur, den)

        pltpu.sync_copy(ostage.at[0], o_hbm[0].at[pl.ds(gnode, _GRP)])
        pltpu.sync_copy(ostage.at[1], o_hbm[1].at[pl.ds(gnode, _GRP)])
        return gcarry

    lax.fori_loop(0, _NGRP, group_body, 0)


@functools.partial(
    pl.kernel,
    out_type=[jax.ShapeDtypeStruct((_N, 128), jnp.int32)] * 2,
    mesh=plsc.VectorSubcoreMesh(core_axis_name="c", subcore_axis_name="s"),
    compiler_params=pltpu.CompilerParams(needs_layout_passes=False),
    scratch_types=[
        pltpu.VMEM((32,), jnp.int32),
        pltpu.VMEM((16,), jnp.int32),
        pltpu.VMEM((2, _GRP, 128), jnp.int32),
        pltpu.VMEM((2, _GRP, 128), jnp.int32),
        pltpu.VMEM((2, 2, _CH, 128), jnp.int32),
        pltpu.VMEM((2, 2, _CH, 128), jnp.int32),
        pltpu.VMEM((2, _CH), jnp.int32),
        pltpu.VMEM((2, _CH + 16), jnp.int32),
        pltpu.VMEM((_D,), _F32),
        pltpu.SemaphoreType.DMA((2,)),
        pltpu.SemaphoreType.DMA((2,)),
    ],
)
def _edge_attn(qa, qb, ka, kb, va, vb, src_hbm, dst_hbm, oa, ob,
               rpbuf, bsbuf, qstage, ostage, kbuf, vbuf, idxbuf, dstbuf,
               accrow, sem_k, sem_v):
    _edge_kernel(qa, qb, ka, kb, va, vb, src_hbm, dst_hbm, oa, ob,
                 rpbuf, bsbuf, qstage, ostage, kbuf, vbuf, idxbuf, dstbuf,
                 accrow, sem_k, sem_v)


def _as_words(x):
    return lax.bitcast_convert_type(x.reshape(_N, _D // 2, 2), jnp.int32)


def _as_bf16(x):
    return lax.bitcast_convert_type(x, _BF).reshape(_N, _D)


def _edge_stage(q, k, v, src, dst):
    qw, kw, vw = _as_words(q), _as_words(k), _as_words(v)
    oa, ob = _edge_attn(qw[:, :128], qw[:, 128:], kw[:, :128], kw[:, 128:],
                        vw[:, :128], vw[:, 128:], src, dst)
    return _as_bf16(jnp.concatenate([oa, ob], axis=1))


def kernel(h, mem, edge_src, edge_dst, inter_src, inter_dst,
           Wq0, Wk0, Wv0, Wo0, Wq1, Wk1, Wv1, Wo1,
           ln0_g, ln0_b, ln1_g, ln1_b, ln2_g, ln2_b,
           W1, b1, W2, b2):
    bf = lambda w: w.astype(_BF)

    q0, k0, v0 = _qkv(h, h, bf(Wq0), bf(Wk0), bf(Wv0))
    o0 = _edge_stage(q0, k0, v0, edge_src, edge_dst)
    k1, v1 = _kv(mem, bf(Wk1), bf(Wv1))
    h1 = _proj_res_ln(o0, bf(Wo0), h, ln0_g, ln0_b)

    q1 = _mm(h1, bf(Wq1))
    o1 = _edge_stage(q1, k1, v1, inter_src, inter_dst)
    h2 = _proj_res_ln(o1, bf(Wo1), h1, ln1_g, ln1_b)

    return _ffn_ln(h2, bf(W1), b1, bf(W2), b2, ln2_g, ln2_b)
